# baseline (speedup 1.0000x reference)
import jax
import jax.numpy as jnp
from jax.experimental import pallas as pl
from jax.experimental.pallas import tpu as pltpu

NUM_TOKENS = 16384
D_MODEL = 2048
NUM_EXPERTS = 16
TOP_K = 2
CHUNK = 512
RING = 8
NCHUNKS = NUM_TOKENS // CHUNK


def _body(x_hbm, w_ref, b_ref, idx_ref, val_ref, bufs, sems):
    def mkdma(c, slot):
        return pltpu.make_async_copy(
            x_hbm.at[pl.ds(c * CHUNK, CHUNK), :], bufs.at[slot], sems.at[slot]
        )

    for c in range(RING):
        mkdma(c, c).start()

    def step(c, _):
        slot = jax.lax.rem(c, RING)
        mkdma(c, slot).wait()
        st = jax.lax.dot_general(
            w_ref[...], bufs[slot], (((1,), (1,)), ((), ())),
            preferred_element_type=jnp.float32,
        )
        nxt = c + RING

        @pl.when(nxt < NCHUNKS)
        def _():
            mkdma(nxt, slot).start()

        st = st + b_ref[...]
        ex = jax.lax.broadcasted_iota(jnp.int32, st.shape, 0)
        m1 = jnp.max(st, axis=0, keepdims=True)
        i1 = jnp.min(jnp.where(st == m1, ex, NUM_EXPERTS), axis=0, keepdims=True)
        s2 = jnp.where(ex == i1, -jnp.inf, st)
        m2 = jnp.max(s2, axis=0, keepdims=True)
        i2 = jnp.min(jnp.where(s2 == m2, ex, NUM_EXPERTS), axis=0, keepdims=True)
        z = jnp.sum(jnp.exp(st - m1), axis=0, keepdims=True)
        v1 = 1.0 / z
        v2 = jnp.exp(m2 - m1) * v1
        col = pl.ds(c * CHUNK, CHUNK)
        idx_ref[:, col] = jnp.concatenate([i1, i2], axis=0)
        val_ref[:, col] = jnp.concatenate([v1, v2], axis=0)
        return 0

    jax.lax.fori_loop(0, NCHUNKS, step, 0)


@jax.jit
def kernel(x, W, b):
    bt = b.reshape(NUM_EXPERTS, 1)
    idx_t, val_t = pl.pallas_call(
        _body,
        in_specs=[
            pl.BlockSpec(memory_space=pltpu.MemorySpace.HBM),
            pl.BlockSpec((NUM_EXPERTS, D_MODEL), lambda: (0, 0)),
            pl.BlockSpec((NUM_EXPERTS, 1), lambda: (0, 0)),
        ],
        out_specs=[
            pl.BlockSpec((TOP_K, NUM_TOKENS), lambda: (0, 0)),
            pl.BlockSpec((TOP_K, NUM_TOKENS), lambda: (0, 0)),
        ],
        out_shape=[
            jax.ShapeDtypeStruct((TOP_K, NUM_TOKENS), jnp.int32),
            jax.ShapeDtypeStruct((TOP_K, NUM_TOKENS), jnp.float32),
        ],
        scratch_shapes=[
            pltpu.VMEM((RING, CHUNK, D_MODEL), jnp.float32),
            pltpu.SemaphoreType.DMA((RING,)),
        ],
    )(x, W, bt)
    return (idx_t.T, val_t.T)


# speedup vs baseline: 1.0701x; 1.0701x over previous
import jax
import jax.numpy as jnp
from jax.experimental import pallas as pl
from jax.experimental.pallas import tpu as pltpu

NUM_TOKENS = 16384
D_MODEL = 2048
NUM_EXPERTS = 16
TOP_K = 2
CHUNK = 256
RING = 4
NCHUNKS = NUM_TOKENS // CHUNK


def _body(x_hbm, w_ref, b_ref, idx_ref, val_ref, bufs, sems):
    def mkdma(c, slot):
        return pltpu.make_async_copy(
            x_hbm.at[pl.ds(c * CHUNK, CHUNK), :], bufs.at[slot], sems.at[slot]
        )

    for c in range(RING):
        mkdma(c, c).start()

    def step(c, _):
        slot = jax.lax.rem(c, RING)
        mkdma(c, slot).wait()
        st = jax.lax.dot_general(
            w_ref[...], bufs[slot], (((1,), (1,)), ((), ())),
            preferred_element_type=jnp.float32,
        )
        nxt = c + RING

        @pl.when(nxt < NCHUNKS)
        def _():
            mkdma(nxt, slot).start()

        st = st + b_ref[...]
        ex = jax.lax.broadcasted_iota(jnp.int32, st.shape, 0)
        m1 = jnp.max(st, axis=0, keepdims=True)
        i1 = jnp.min(jnp.where(st == m1, ex, NUM_EXPERTS), axis=0, keepdims=True)
        s2 = jnp.where(ex == i1, -jnp.inf, st)
        m2 = jnp.max(s2, axis=0, keepdims=True)
        i2 = jnp.min(jnp.where(s2 == m2, ex, NUM_EXPERTS), axis=0, keepdims=True)
        z = jnp.sum(jnp.exp(st - m1), axis=0, keepdims=True)
        v1 = 1.0 / z
        v2 = jnp.exp(m2 - m1) * v1
        col = pl.ds(c * CHUNK, CHUNK)
        idx_ref[:, col] = jnp.concatenate([i1, i2], axis=0)
        val_ref[:, col] = jnp.concatenate([v1, v2], axis=0)
        return 0

    jax.lax.fori_loop(0, NCHUNKS, step, 0)


@jax.jit
def kernel(x, W, b):
    bt = b.reshape(NUM_EXPERTS, 1)
    idx_t, val_t = pl.pallas_call(
        _body,
        in_specs=[
            pl.BlockSpec(memory_space=pltpu.MemorySpace.HBM),
            pl.BlockSpec((NUM_EXPERTS, D_MODEL), lambda: (0, 0)),
            pl.BlockSpec((NUM_EXPERTS, 1), lambda: (0, 0)),
        ],
        out_specs=[
            pl.BlockSpec((TOP_K, NUM_TOKENS), lambda: (0, 0)),
            pl.BlockSpec((TOP_K, NUM_TOKENS), lambda: (0, 0)),
        ],
        out_shape=[
            jax.ShapeDtypeStruct((TOP_K, NUM_TOKENS), jnp.int32),
            jax.ShapeDtypeStruct((TOP_K, NUM_TOKENS), jnp.float32),
        ],
        scratch_shapes=[
            pltpu.VMEM((RING, CHUNK, D_MODEL), jnp.float32),
            pltpu.SemaphoreType.DMA((RING,)),
        ],
    )(x, W, bt)
    return (idx_t.T, val_t.T)


# speedup vs baseline: 1.0800x; 1.0092x over previous
import jax
import jax.numpy as jnp
from jax.experimental import pallas as pl
from jax.experimental.pallas import tpu as pltpu

NUM_TOKENS = 16384
D_MODEL = 2048
NUM_EXPERTS = 16
TOP_K = 2
CHUNK = 128
RING = 6
NCHUNKS = NUM_TOKENS // CHUNK


def _body(x_hbm, w_ref, b_ref, idx_ref, val_ref, bufs, sems):
    def mkdma(c, slot):
        return pltpu.make_async_copy(
            x_hbm.at[pl.ds(c * CHUNK, CHUNK), :], bufs.at[slot], sems.at[slot]
        )

    for c in range(RING):
        mkdma(c, c).start()

    def step(c, _):
        slot = jax.lax.rem(c, RING)
        mkdma(c, slot).wait()
        st = jax.lax.dot_general(
            w_ref[...], bufs[slot], (((1,), (1,)), ((), ())),
            preferred_element_type=jnp.float32,
        )
        nxt = c + RING

        @pl.when(nxt < NCHUNKS)
        def _():
            mkdma(nxt, slot).start()

        st = st + b_ref[...]
        ex = jax.lax.broadcasted_iota(jnp.int32, st.shape, 0)
        m1 = jnp.max(st, axis=0, keepdims=True)
        i1 = jnp.min(jnp.where(st == m1, ex, NUM_EXPERTS), axis=0, keepdims=True)
        s2 = jnp.where(ex == i1, -jnp.inf, st)
        m2 = jnp.max(s2, axis=0, keepdims=True)
        i2 = jnp.min(jnp.where(s2 == m2, ex, NUM_EXPERTS), axis=0, keepdims=True)
        z = jnp.sum(jnp.exp(st - m1), axis=0, keepdims=True)
        v1 = 1.0 / z
        v2 = jnp.exp(m2 - m1) * v1
        col = pl.ds(c * CHUNK, CHUNK)
        idx_ref[:, col] = jnp.concatenate([i1, i2], axis=0)
        val_ref[:, col] = jnp.concatenate([v1, v2], axis=0)
        return 0

    jax.lax.fori_loop(0, NCHUNKS, step, 0)


@jax.jit
def kernel(x, W, b):
    bt = b.reshape(NUM_EXPERTS, 1)
    idx_t, val_t = pl.pallas_call(
        _body,
        in_specs=[
            pl.BlockSpec(memory_space=pltpu.MemorySpace.HBM),
            pl.BlockSpec((NUM_EXPERTS, D_MODEL), lambda: (0, 0)),
            pl.BlockSpec((NUM_EXPERTS, 1), lambda: (0, 0)),
        ],
        out_specs=[
            pl.BlockSpec((TOP_K, NUM_TOKENS), lambda: (0, 0)),
            pl.BlockSpec((TOP_K, NUM_TOKENS), lambda: (0, 0)),
        ],
        out_shape=[
            jax.ShapeDtypeStruct((TOP_K, NUM_TOKENS), jnp.int32),
            jax.ShapeDtypeStruct((TOP_K, NUM_TOKENS), jnp.float32),
        ],
        scratch_shapes=[
            pltpu.VMEM((RING, CHUNK, D_MODEL), jnp.float32),
            pltpu.SemaphoreType.DMA((RING,)),
        ],
    )(x, W, bt)
    return (idx_t.T, val_t.T)


# speedup vs baseline: 1.0832x; 1.0029x over previous
import jax
import jax.numpy as jnp
from jax.experimental import pallas as pl
from jax.experimental.pallas import tpu as pltpu

NUM_TOKENS = 16384
D_MODEL = 2048
NUM_EXPERTS = 16
TOP_K = 2
CHUNK = 256
RING = 4
NCHUNKS = NUM_TOKENS // CHUNK


def _body(x_hbm, w_ref, b_ref, idx_ref, val_ref, bufs, sems):
    def mkdma(c, slot):
        return pltpu.make_async_copy(
            x_hbm.at[pl.ds(c * CHUNK, CHUNK), :], bufs.at[slot], sems.at[slot]
        )

    for c in range(RING):
        mkdma(c, c).start()

    def step(c, _):
        slot = jax.lax.rem(c, RING)
        mkdma(c, slot).wait()
        st = jax.lax.dot_general(
            w_ref[...], bufs[slot], (((1,), (1,)), ((), ())),
            preferred_element_type=jnp.float32,
        )
        nxt = c + RING

        @pl.when(nxt < NCHUNKS)
        def _():
            mkdma(nxt, slot).start()

        st = st + b_ref[...]
        ex = jax.lax.broadcasted_iota(jnp.int32, st.shape, 0)
        m1 = jnp.max(st, axis=0, keepdims=True)
        i1 = jnp.min(jnp.where(st == m1, ex, NUM_EXPERTS), axis=0, keepdims=True)
        s2 = jnp.where(ex == i1, -jnp.inf, st)
        m2 = jnp.max(s2, axis=0, keepdims=True)
        i2 = jnp.min(jnp.where(s2 == m2, ex, NUM_EXPERTS), axis=0, keepdims=True)
        z = jnp.sum(jnp.exp(st - m1), axis=0, keepdims=True)
        v1 = 1.0 / z
        v2 = jnp.exp(m2 - m1) * v1
        col = pl.ds(c * CHUNK, CHUNK)
        idx_ref[:, col] = jnp.concatenate([i1, i2], axis=0)
        val_ref[:, col] = jnp.concatenate([v1, v2], axis=0)
        return 0

    jax.lax.fori_loop(0, NCHUNKS, step, 0)


@jax.jit
def kernel(x, W, b):
    bt = b.reshape(NUM_EXPERTS, 1)
    idx_t, val_t = pl.pallas_call(
        _body,
        in_specs=[
            pl.BlockSpec(memory_space=pltpu.MemorySpace.HBM),
            pl.BlockSpec((NUM_EXPERTS, D_MODEL), lambda: (0, 0)),
            pl.BlockSpec((NUM_EXPERTS, 1), lambda: (0, 0)),
        ],
        out_specs=[
            pl.BlockSpec((TOP_K, NUM_TOKENS), lambda: (0, 0)),
            pl.BlockSpec((TOP_K, NUM_TOKENS), lambda: (0, 0)),
        ],
        out_shape=[
            jax.ShapeDtypeStruct((TOP_K, NUM_TOKENS), jnp.int32),
            jax.ShapeDtypeStruct((TOP_K, NUM_TOKENS), jnp.float32),
        ],
        scratch_shapes=[
            pltpu.VMEM((RING, CHUNK, D_MODEL), jnp.float32),
            pltpu.SemaphoreType.DMA((RING,)),
        ],
    )(x, W, bt)
    return (idx_t.T, val_t.T)
